# baseline (speedup 1.0000x reference)
import functools

import jax
import jax.numpy as jnp
from jax import lax
from jax.experimental import pallas as pl
from jax.experimental.pallas import tpu as pltpu
from jax.experimental.pallas import tpu_sc as plsc

_RPT = 128


def _tables_kernel(xs_ref, ys_ref, xyp_ref, dx2_ref, dy2_ref):
    px = xyp_ref[0, :, 0:1]
    py = xyp_ref[0, :, 1:2]
    ddx = xs_ref[0, 0:1, :] - px
    dx2_ref[0] = ddx * ddx
    ddy = ys_ref[0, 0:1, :] - py
    dy2 = ddy * ddy
    for t in range(dy2_ref.shape[1]):
        dy2_ref[0, t] = dy2[:, t * _RPT:(t + 1) * _RPT]


def _minplus_idx_kernel(dx2_ref, dy2_ref, out_ref):
    b = pl.program_id(0)
    n = dx2_ref.shape[1]
    w = dx2_ref.shape[2]
    dx2 = dx2_ref[0]
    niota = jax.lax.broadcasted_iota(jnp.int32, (n, w), 0) + b * n
    big = jnp.int32(2 ** 30)
    for rr in range(_RPT):
        d2 = dx2 + dy2_ref[0, 0, :, rr:rr + 1]
        m = jnp.min(d2, axis=0, keepdims=True)
        idx = jnp.min(jnp.where(d2 == m, niota, big), axis=0, keepdims=True)
        out_ref[0, 0, :, rr * w:(rr + 1) * w] = idx


_CHUNK = 512


def _sc_gather_kernel(table_hbm, idx_hbm, out_hbm, idx_v, rows_v, sem, *,
                      qpw, num_cores):
    wid = lax.axis_index("s") * num_cores + lax.axis_index("c")
    base = wid * qpw
    pltpu.sync_copy(idx_hbm.at[pl.ds(base, qpw)], idx_v)
    for k in range(qpw // _CHUNK):
        idx_c = idx_v.at[pl.ds(k * _CHUNK, _CHUNK)]
        pltpu.async_copy(table_hbm.at[idx_c], rows_v, sem).wait()
        pltpu.sync_copy(rows_v, out_hbm.at[pl.ds(base + k * _CHUNK, _CHUNK)])


def kernel(R_pc, XY_pc, XY_grd):
    B, C, N = R_pc.shape
    Q = XY_grd.shape[2]
    H = Wd = int(round(Q ** 0.5))
    xs = XY_grd[:, 0, :Wd].reshape(B, 1, Wd)
    ys = XY_grd[:, 1, ::Wd].reshape(B, 1, H)
    XY_pcT = XY_pc.transpose(0, 2, 1)
    NT = H // _RPT

    dx2, dy2 = pl.pallas_call(
        _tables_kernel,
        grid=(B,),
        in_specs=[
            pl.BlockSpec((1, 1, Wd), lambda b: (b, 0, 0)),
            pl.BlockSpec((1, 1, H), lambda b: (b, 0, 0)),
            pl.BlockSpec((1, N, 2), lambda b: (b, 0, 0)),
        ],
        out_specs=[
            pl.BlockSpec((1, N, Wd), lambda b: (b, 0, 0)),
            pl.BlockSpec((1, NT, N, _RPT), lambda b: (b, 0, 0, 0)),
        ],
        out_shape=[
            jax.ShapeDtypeStruct((B, N, Wd), jnp.float32),
            jax.ShapeDtypeStruct((B, NT, N, _RPT), jnp.float32),
        ],
        compiler_params=pltpu.CompilerParams(
            dimension_semantics=(pltpu.PARALLEL,)),
    )(xs, ys, XY_pcT)

    idx = pl.pallas_call(
        _minplus_idx_kernel,
        grid=(B, NT),
        in_specs=[
            pl.BlockSpec((1, N, Wd), lambda b, t: (b, 0, 0)),
            pl.BlockSpec((1, 1, N, _RPT), lambda b, t: (b, t, 0, 0)),
        ],
        out_specs=pl.BlockSpec((1, 1, 1, _RPT * Wd), lambda b, t: (b, t, 0, 0)),
        out_shape=jax.ShapeDtypeStruct((B, NT, 1, _RPT * Wd), jnp.int32),
        compiler_params=pltpu.CompilerParams(
            dimension_semantics=(pltpu.PARALLEL, pltpu.PARALLEL)),
    )(dx2, dy2)

    info = plsc.get_sparse_core_info()
    nw = info.num_cores * info.num_subcores
    qpw = (B * Q) // nw
    table = jnp.pad(R_pc.transpose(0, 2, 1).reshape(B * N, C),
                    ((0, 0), (0, 128 - C)))
    idx_flat = idx.reshape(B * Q)

    sc_gather = functools.partial(
        _sc_gather_kernel, qpw=qpw, num_cores=info.num_cores)
    gathered = pl.kernel(
        sc_gather,
        mesh=plsc.VectorSubcoreMesh(core_axis_name="c", subcore_axis_name="s"),
        out_type=jax.ShapeDtypeStruct((B * Q, 128), jnp.float32),
        scratch_types=[
            pltpu.VMEM((qpw,), jnp.int32),
            pltpu.VMEM((_CHUNK, 128), jnp.float32),
            pltpu.SemaphoreType.DMA,
        ],
    )(table, idx_flat)

    return (gathered.reshape(B, Q, 128)[:, :, :C]
            .transpose(0, 2, 1).reshape(B, C, H, Wd))


# speedup vs baseline: 1.1936x; 1.1936x over previous
import functools

import jax
import jax.numpy as jnp
from jax import lax
from jax.experimental import pallas as pl
from jax.experimental.pallas import tpu as pltpu
from jax.experimental.pallas import tpu_sc as plsc

_RPT = 64


def _tables_kernel(xs_ref, ys_ref, xyp_ref, dx2_ref, dy2_ref):
    px = xyp_ref[0, :, 0:1]
    py = xyp_ref[0, :, 1:2]
    ddx = xs_ref[0, 0:1, :] - px
    dx2_ref[0] = ddx * ddx
    ddy = ys_ref[0, 0:1, :] - py
    dy2 = ddy * ddy
    for t in range(dy2_ref.shape[1]):
        dy2_ref[0, t] = dy2[:, t * _RPT:(t + 1) * _RPT]


def _minplus_idx_kernel(dx2_ref, dy2_ref, out_ref):
    b = pl.program_id(0)
    n = dx2_ref.shape[1]
    w = dx2_ref.shape[2]
    dx2 = dx2_ref[0]
    niota = jax.lax.broadcasted_iota(jnp.int32, (n, w), 0) + b * n
    big = jnp.int32(2 ** 30)
    for rr in range(_RPT):
        d2 = dx2 + dy2_ref[0, 0, :, rr:rr + 1]
        m = jnp.min(d2, axis=0, keepdims=True)
        idx = jnp.min(jnp.where(d2 == m, niota, big), axis=0, keepdims=True)
        out_ref[0, 0, :, rr * w:(rr + 1) * w] = idx


_CHUNK = 512


def _sc_gather_kernel(table_hbm, idx_hbm, out_hbm, idx_v, rows_v, sem, *,
                      qpw, num_cores):
    wid = lax.axis_index("s") * num_cores + lax.axis_index("c")
    base = wid * qpw
    pltpu.sync_copy(idx_hbm.at[pl.ds(base, qpw)], idx_v)
    for k in range(qpw // _CHUNK):
        idx_c = idx_v.at[pl.ds(k * _CHUNK, _CHUNK)]
        pltpu.async_copy(table_hbm.at[idx_c], rows_v, sem).wait()
        pltpu.sync_copy(rows_v, out_hbm.at[pl.ds(base + k * _CHUNK, _CHUNK)])


def kernel(R_pc, XY_pc, XY_grd):
    B, C, N = R_pc.shape
    Q = XY_grd.shape[2]
    H = Wd = int(round(Q ** 0.5))
    xs = XY_grd[:, 0, :Wd].reshape(B, 1, Wd)
    ys = XY_grd[:, 1, ::Wd].reshape(B, 1, H)
    XY_pcT = XY_pc.transpose(0, 2, 1)
    NT = H // _RPT

    dx2, dy2 = pl.pallas_call(
        _tables_kernel,
        grid=(B,),
        in_specs=[
            pl.BlockSpec((1, 1, Wd), lambda b: (b, 0, 0)),
            pl.BlockSpec((1, 1, H), lambda b: (b, 0, 0)),
            pl.BlockSpec((1, N, 2), lambda b: (b, 0, 0)),
        ],
        out_specs=[
            pl.BlockSpec((1, N, Wd), lambda b: (b, 0, 0)),
            pl.BlockSpec((1, NT, N, _RPT), lambda b: (b, 0, 0, 0)),
        ],
        out_shape=[
            jax.ShapeDtypeStruct((B, N, Wd), jnp.float32),
            jax.ShapeDtypeStruct((B, NT, N, _RPT), jnp.float32),
        ],
        compiler_params=pltpu.CompilerParams(
            dimension_semantics=(pltpu.PARALLEL,)),
    )(xs, ys, XY_pcT)

    idx = pl.pallas_call(
        _minplus_idx_kernel,
        grid=(B, NT),
        in_specs=[
            pl.BlockSpec((1, N, Wd), lambda b, t: (b, 0, 0)),
            pl.BlockSpec((1, 1, N, _RPT), lambda b, t: (b, t, 0, 0)),
        ],
        out_specs=pl.BlockSpec((1, 1, 1, _RPT * Wd), lambda b, t: (b, t, 0, 0)),
        out_shape=jax.ShapeDtypeStruct((B, NT, 1, _RPT * Wd), jnp.int32),
        compiler_params=pltpu.CompilerParams(
            dimension_semantics=(pltpu.PARALLEL, pltpu.PARALLEL)),
    )(dx2, dy2)

    info = plsc.get_sparse_core_info()
    nw = info.num_cores * info.num_subcores
    qpw = (B * Q) // nw
    table = jnp.pad(R_pc.transpose(0, 2, 1).reshape(B * N, C),
                    ((0, 0), (0, 128 - C)))
    idx_flat = idx.reshape(B * Q)

    sc_gather = functools.partial(
        _sc_gather_kernel, qpw=qpw, num_cores=info.num_cores)
    gathered = pl.kernel(
        sc_gather,
        mesh=plsc.VectorSubcoreMesh(core_axis_name="c", subcore_axis_name="s"),
        out_type=jax.ShapeDtypeStruct((B * Q, 128), jnp.float32),
        scratch_types=[
            pltpu.VMEM((qpw,), jnp.int32),
            pltpu.VMEM((_CHUNK, 128), jnp.float32),
            pltpu.SemaphoreType.DMA,
        ],
    )(table, idx_flat)

    return (gathered.reshape(B, Q, 128)[:, :, :C]
            .transpose(0, 2, 1).reshape(B, C, H, Wd))
